# baseline (speedup 1.0000x reference)
import functools

import jax
import jax.numpy as jnp
from jax import lax
from jax.experimental import pallas as pl
from jax.experimental.pallas import tpu as pltpu
from jax.experimental.pallas import tpu_sc as plsc

VOCAB_ROWS = 49408
N_CLS = 1024
SEQ_LEN = 77
CTX_DIM = 512
N_CTX = 8

N_CLS_SC = 768
N_CLS_TC = N_CLS - N_CLS_SC

_info = plsc.get_sparse_core_info()
_NC = _info.num_cores
_NS = _info.num_subcores
_NW = _NC * _NS
_B = N_CLS_SC * SEQ_LEN
_RPW = _B // _NW
_CHUNK = 88
_NCHUNK = _RPW // _CHUNK


def _sc_body(idx_hbm, table_hbm, out_hbm, idx_v, rows0, rows1, sem0, sem1):
    wid = lax.axis_index("s") * _NC + lax.axis_index("c")
    base = wid * _RPW

    pltpu.sync_copy(idx_hbm.at[wid], idx_v)

    pltpu.async_copy(table_hbm.at[idx_v.at[0]], rows0, sem0)

    @pl.loop(0, _NCHUNK, step=2)
    def step(j):
        @pl.when(j + 1 < _NCHUNK)
        def _():
            pltpu.async_copy(table_hbm.at[idx_v.at[j + 1]], rows1, sem1)

        pltpu.make_async_copy(table_hbm.at[idx_v.at[j]], rows0, sem0).wait()
        pltpu.sync_copy(rows0, out_hbm.at[pl.ds(base + j * _CHUNK, _CHUNK)])

        @pl.when(j + 2 < _NCHUNK)
        def _():
            pltpu.async_copy(table_hbm.at[idx_v.at[j + 2]], rows0, sem0)

        @pl.when(j + 1 < _NCHUNK)
        def _():
            pltpu.make_async_copy(
                table_hbm.at[idx_v.at[j + 1]], rows1, sem1).wait()
            pltpu.sync_copy(
                rows1, out_hbm.at[pl.ds(base + (j + 1) * _CHUNK, _CHUNK)])


def _tc_body(idx_ref, table_ref, out_ref, sem):
    c = pl.program_id(0)
    copies = [
        pltpu.make_async_copy(
            table_ref.at[pl.ds(idx_ref[c * SEQ_LEN + i], 1)],
            out_ref.at[0, pl.ds(i, 1)],
            sem,
        )
        for i in range(SEQ_LEN)
    ]
    for cp in copies:
        cp.start()
    for cp in copies:
        cp.wait()


def kernel(tokens, table, ctx):
    tbl2 = jnp.concatenate([table, ctx], axis=0)
    ctx_ids = jnp.broadcast_to(
        jnp.arange(VOCAB_ROWS, VOCAB_ROWS + N_CTX, dtype=jnp.int32)[None, :],
        (N_CLS, N_CTX))
    idx = jnp.concatenate(
        [tokens[:, :1], ctx_ids, tokens[:, 1 + N_CTX:]], axis=1)

    idx_sc = idx[:N_CLS_SC].reshape(_NW, _NCHUNK, _CHUNK)
    idx_tc = idx[N_CLS_SC:]

    f_sc = pl.kernel(
        _sc_body,
        out_type=jax.ShapeDtypeStruct((_B, CTX_DIM), jnp.float32),
        mesh=plsc.VectorSubcoreMesh(core_axis_name="c", subcore_axis_name="s"),
        scratch_types=[
            pltpu.VMEM((_NCHUNK, _CHUNK), jnp.int32),
            pltpu.VMEM((_CHUNK, CTX_DIM), jnp.float32),
            pltpu.VMEM((_CHUNK, CTX_DIM), jnp.float32),
            pltpu.SemaphoreType.DMA,
            pltpu.SemaphoreType.DMA,
        ],
    )

    f_tc = pl.pallas_call(
        _tc_body,
        grid_spec=pltpu.PrefetchScalarGridSpec(
            num_scalar_prefetch=1,
            grid=(N_CLS_TC,),
            in_specs=[pl.BlockSpec(memory_space=pl.ANY)],
            out_specs=pl.BlockSpec((1, SEQ_LEN, CTX_DIM),
                                   lambda c, idx: (c, 0, 0)),
            scratch_shapes=[pltpu.SemaphoreType.DMA],
        ),
        out_shape=jax.ShapeDtypeStruct((N_CLS_TC, SEQ_LEN, CTX_DIM),
                                       jnp.float32),
        compiler_params=pltpu.CompilerParams(
            dimension_semantics=("arbitrary",)),
    )

    out_sc = f_sc(idx_sc, tbl2).reshape(N_CLS_SC, SEQ_LEN, CTX_DIM)
    out_tc = f_tc(idx_tc.reshape(-1), tbl2)
    return jnp.concatenate([out_sc, out_tc], axis=0)


# speedup vs baseline: 1.5862x; 1.5862x over previous
import jax
import jax.numpy as jnp
from jax import lax
from jax.experimental import pallas as pl
from jax.experimental.pallas import tpu as pltpu
from jax.experimental.pallas import tpu_sc as plsc

VOCAB_ROWS = 49408
N_CLS = 1024
SEQ_LEN = 77
CTX_DIM = 512
N_CTX = 8

_info = plsc.get_sparse_core_info()
_NC = _info.num_cores
_NS = _info.num_subcores
_NW = _NC * _NS
_B = N_CLS * SEQ_LEN
_RPW = _B // _NW
_CHUNK = 112
_NCHUNK = _RPW // _CHUNK


def _body(idx_hbm, table_hbm, out_hbm, idx_v, rows0, rows1, sem0, sem1):
    wid = lax.axis_index("s") * _NC + lax.axis_index("c")
    base = wid * _RPW

    pltpu.sync_copy(idx_hbm.at[wid], idx_v)

    pltpu.async_copy(table_hbm.at[idx_v.at[0]], rows0, sem0)

    @pl.loop(0, _NCHUNK, step=2)
    def step(j):
        pltpu.async_copy(table_hbm.at[idx_v.at[j + 1]], rows1, sem1)
        pltpu.make_async_copy(table_hbm.at[idx_v.at[j]], rows0, sem0).wait()
        pltpu.sync_copy(rows0, out_hbm.at[pl.ds(base + j * _CHUNK, _CHUNK)])

        @pl.when(j + 2 < _NCHUNK)
        def _():
            pltpu.async_copy(table_hbm.at[idx_v.at[j + 2]], rows0, sem0)

        pltpu.make_async_copy(table_hbm.at[idx_v.at[j + 1]], rows1, sem1).wait()
        pltpu.sync_copy(
            rows1, out_hbm.at[pl.ds(base + (j + 1) * _CHUNK, _CHUNK)])


def kernel(tokens, table, ctx):
    tbl2 = jnp.concatenate([table, ctx], axis=0)
    ctx_ids = jnp.broadcast_to(
        jnp.arange(VOCAB_ROWS, VOCAB_ROWS + N_CTX, dtype=jnp.int32)[None, :],
        (N_CLS, N_CTX))
    idx = jnp.concatenate(
        [tokens[:, :1], ctx_ids, tokens[:, 1 + N_CTX:]], axis=1)
    idx3 = idx.reshape(_NW, _NCHUNK, _CHUNK)
    f = pl.kernel(
        _body,
        out_type=jax.ShapeDtypeStruct((_B, CTX_DIM), jnp.float32),
        mesh=plsc.VectorSubcoreMesh(core_axis_name="c", subcore_axis_name="s"),
        scratch_types=[
            pltpu.VMEM((_NCHUNK, _CHUNK), jnp.int32),
            pltpu.VMEM((_CHUNK, CTX_DIM), jnp.float32),
            pltpu.VMEM((_CHUNK, CTX_DIM), jnp.float32),
            pltpu.SemaphoreType.DMA,
            pltpu.SemaphoreType.DMA,
        ],
    )
    return f(idx3, tbl2).reshape(N_CLS, SEQ_LEN, CTX_DIM)
